# baseline (speedup 1.0000x reference)
import functools

import jax
import jax.numpy as jnp
from jax import lax
from jax.experimental import pallas as pl
from jax.experimental.pallas import tpu as pltpu
from jax.experimental.pallas import tpu_sc as plsc

EXPERTS = 8
TOPK = 2
TB1 = 256
SB = 256
STEPS = 40


def _route_body(x_ref, wg_ref, bg_ref, logits_ref, idx_ref, wn_ref,
                within_ref, counts_ref, carry_ref):
    b = pl.program_id(0)

    @pl.when(b == 0)
    def _():
        carry_ref[...] = jnp.zeros_like(carry_ref)

    x = x_ref[...]
    logits = jax.lax.dot_general(
        x, wg_ref[...],
        dimension_numbers=(((1,), (1,)), ((), ())),
        preferred_element_type=jnp.float32,
    ) + bg_ref[...]
    logits_ref[...] = logits

    cols = [logits[:, e:e + 1] for e in range(EXPERTS)]
    lmax = cols[0]
    for e in range(1, EXPERTS):
        lmax = jnp.maximum(lmax, cols[e])

    ranks = []
    for e in range(EXPERTS):
        rank = None
        for j in range(EXPERTS):
            if j == e:
                continue
            beats = (cols[j] >= cols[e]) if j < e else (cols[j] > cols[e])
            bf = beats.astype(jnp.float32)
            rank = bf if rank is None else rank + bf
        ranks.append(rank)

    denom = None
    exps = []
    for e in range(EXPERTS):
        we = jnp.where(ranks[e] < 2.0, jnp.exp(cols[e] - lmax), 0.0)
        exps.append(we)
        denom = we if denom is None else denom + we

    sels = []
    for k in range(TOPK):
        sel_idx = None
        sel_w = None
        for e in range(EXPERTS):
            m = (ranks[e] == jnp.float32(k)).astype(jnp.float32)
            ide = m * jnp.float32(e)
            wde = m * exps[e]
            sel_idx = ide if sel_idx is None else sel_idx + ide
            sel_w = wde if sel_w is None else sel_w + wde
        sels.append((sel_idx, sel_w))

    idx_ref[...] = jnp.concatenate(
        [s[0] for s in sels], axis=1).astype(jnp.int32)
    wn_ref[...] = jnp.concatenate([s[1] for s in sels], axis=1) / denom

    sel_all = jnp.concatenate([s[0] for s in sels], axis=0)
    onehot = (sel_all == jax.lax.broadcasted_iota(
        jnp.int32, (TOPK * TB1, EXPERTS), 1).astype(jnp.float32)
    ).astype(jnp.float32)
    n2 = TOPK * TB1
    r_i = jax.lax.broadcasted_iota(jnp.int32, (n2, n2), 0)
    c_i = jax.lax.broadcasted_iota(jnp.int32, (n2, n2), 1)
    ltri = (r_i > c_i).astype(jnp.float32)
    prefix = jax.lax.dot_general(
        ltri, onehot,
        dimension_numbers=(((1,), (0,)), ((), ())),
        preferred_element_type=jnp.float32,
    ) + carry_ref[...]
    within = jnp.sum(onehot * prefix, axis=1, keepdims=True)
    within_ref[...] = jnp.concatenate(
        [within[:TB1], within[TB1:]], axis=1).astype(jnp.int32)

    carry_ref[...] += jnp.sum(onehot, axis=0, keepdims=True)
    counts_ref[...] = carry_ref[...]


def _ragged_mm_body(tbl_ref, hs_ref, we_ref, be_ref, out_ref):
    s = pl.program_id(0)
    e = tbl_ref[1, s]
    r0 = tbl_ref[2, s]
    r1 = tbl_ref[3, s]
    init = tbl_ref[4, s]
    tile = tbl_ref[0, s]

    w_e = jnp.reshape(we_ref[pl.ds(e, 1), :, :], (we_ref.shape[1],
                                                  we_ref.shape[2]))
    b_e = be_ref[pl.ds(e, 1), :]
    y = jax.lax.dot_general(
        hs_ref[...].astype(jnp.bfloat16), w_e,
        dimension_numbers=(((1,), (1,)), ((), ())),
        preferred_element_type=jnp.float32,
    ) + b_e
    row = tile * SB + jax.lax.broadcasted_iota(jnp.int32, (SB, 1), 0)
    mask = jnp.logical_and(row >= r0, row < r1)
    contrib = jnp.where(mask, y, 0.0)

    @pl.when(init == 1)
    def _():
        out_ref[...] = contrib

    @pl.when(init != 1)
    def _():
        out_ref[...] += contrib


def _route(h, Wg, bg2, T, H, E):
    nb = T // TB1
    return pl.pallas_call(
        _route_body,
        grid=(nb,),
        in_specs=[
            pl.BlockSpec((TB1, H), lambda b: (b, 0)),
            pl.BlockSpec((E, H), lambda b: (0, 0)),
            pl.BlockSpec((1, E), lambda b: (0, 0)),
        ],
        out_specs=[
            pl.BlockSpec((TB1, E), lambda b: (b, 0)),
            pl.BlockSpec((TB1, TOPK), lambda b: (b, 0)),
            pl.BlockSpec((TB1, TOPK), lambda b: (b, 0)),
            pl.BlockSpec((TB1, TOPK), lambda b: (b, 0)),
            pl.BlockSpec((1, E), lambda b: (0, 0)),
        ],
        out_shape=[
            jax.ShapeDtypeStruct((T, E), jnp.float32),
            jax.ShapeDtypeStruct((T, TOPK), jnp.int32),
            jax.ShapeDtypeStruct((T, TOPK), jnp.float32),
            jax.ShapeDtypeStruct((T, TOPK), jnp.int32),
            jax.ShapeDtypeStruct((1, E), jnp.float32),
        ],
        scratch_shapes=[pltpu.VMEM((1, E), jnp.float32)],
        compiler_params=pltpu.CompilerParams(
            dimension_semantics=("arbitrary",)),
    )(h, Wg, bg2)


def _step_table(off9, A):
    nt = A // SB
    t_ids = jnp.arange(nt, dtype=jnp.int32)[:, None]
    lo = off9[:-1][None, :]
    hi = off9[1:][None, :]
    r0 = jnp.maximum(lo, t_ids * SB)
    r1 = jnp.minimum(hi, (t_ids + 1) * SB)
    active = (r1 > r0).reshape(-1)
    e_f = jnp.broadcast_to(jnp.arange(EXPERTS, dtype=jnp.int32)[None, :],
                           (nt, EXPERTS)).reshape(-1)
    t_f = jnp.broadcast_to(t_ids, (nt, EXPERTS)).reshape(-1)
    r0_f = r0.reshape(-1)
    r1_f = r1.reshape(-1)
    init_f = jnp.logical_and(active, r0_f == t_f * SB).astype(jnp.int32)

    dest = jnp.cumsum(active.astype(jnp.int32)) - active.astype(jnp.int32)
    dest = jnp.where(active, dest, STEPS - 1)
    pad = jnp.array([nt - 1, 0, 0, 0, 0], dtype=jnp.int32)
    vals = jnp.stack([t_f, e_f, r0_f, r1_f, init_f], axis=0)
    vals = jnp.where(active[None, :], vals, pad[:, None])
    tbl = jnp.zeros((5, STEPS), dtype=jnp.int32)
    tbl = tbl.at[0, :].set(nt - 1)
    tbl = tbl.at[:, dest].set(vals)
    return tbl


def _ragged_mm(tbl, hs, We, be, A, H, E):
    grid_spec = pltpu.PrefetchScalarGridSpec(
        num_scalar_prefetch=1,
        grid=(STEPS,),
        in_specs=[
            pl.BlockSpec((SB, H), lambda s, tbl: (tbl[0, s], 0)),
            pl.BlockSpec((E, H, H), lambda s, tbl: (0, 0, 0)),
            pl.BlockSpec((E, H), lambda s, tbl: (0, 0)),
        ],
        out_specs=pl.BlockSpec((SB, H), lambda s, tbl: (tbl[0, s], 0)),
    )
    return pl.pallas_call(
        _ragged_mm_body,
        grid_spec=grid_spec,
        out_shape=jax.ShapeDtypeStruct((A, H), jnp.float32),
        compiler_params=pltpu.CompilerParams(
            dimension_semantics=("arbitrary",)),
    )(tbl, hs, We, be)


_NW = 32
_DCH = 64
_CCH = 32


def _dispatch_sc(h, posA, posB, T, H, A):
    per_t = T // _NW
    nch = per_t // _DCH
    mesh = plsc.VectorSubcoreMesh(core_axis_name="c", subcore_axis_name="s")

    @functools.partial(
        pl.kernel, mesh=mesh,
        out_type=jax.ShapeDtypeStruct((A, H), jnp.float32),
        scratch_types=[
            pltpu.VMEM((_DCH, H), jnp.float32),
            pltpu.VMEM((_DCH, H), jnp.float32),
            pltpu.VMEM((_DCH,), jnp.int32),
            pltpu.VMEM((_DCH,), jnp.int32),
            pltpu.VMEM((_DCH,), jnp.int32),
            pltpu.VMEM((_DCH,), jnp.int32),
            pltpu.SemaphoreType.DMA,
            pltpu.SemaphoreType.DMA,
        ],
    )
    def k(h_hbm, posA_hbm, posB_hbm, hs_hbm,
          buf0, buf1, ia0, ia1, ib0, ib1, sg, ss):
        wid = lax.axis_index("s") * 2 + lax.axis_index("c")
        t0 = wid * per_t
        bufs = (buf0, buf1)
        ias = (ia0, ia1)
        ibs = (ib0, ib1)
        gets = [None] * nch
        gets[0] = pltpu.async_copy(h_hbm.at[pl.ds(t0, _DCH)], bufs[0], sg)
        puts = []
        for c in range(nch):
            base = t0 + c * _DCH
            pltpu.sync_copy(posA_hbm.at[pl.ds(base, _DCH)], ias[c % 2])
            pltpu.sync_copy(posB_hbm.at[pl.ds(base, _DCH)], ibs[c % 2])
            gets[c].wait()
            if c + 1 < nch:
                gets[c + 1] = pltpu.async_copy(
                    h_hbm.at[pl.ds(t0 + (c + 1) * _DCH, _DCH)],
                    bufs[(c + 1) % 2], sg)
            puts.append(pltpu.async_copy(
                bufs[c % 2], hs_hbm.at[ias[c % 2]], ss))
            puts.append(pltpu.async_copy(
                bufs[c % 2], hs_hbm.at[ibs[c % 2]], ss))
        for hdl in puts:
            hdl.wait()

    return k(h, posA, posB)


def _combine_sc(out_sorted, posA, posB, wA, wB, T, H):
    per_t = T // _NW
    nch = per_t // _CCH
    mesh = plsc.VectorSubcoreMesh(core_axis_name="c", subcore_axis_name="s")

    @functools.partial(
        pl.kernel, mesh=mesh,
        out_type=jax.ShapeDtypeStruct((T, H), jnp.float32),
        scratch_types=[
            pltpu.VMEM((_CCH, H), jnp.float32),
            pltpu.VMEM((_CCH, H), jnp.float32),
            pltpu.VMEM((_CCH, H), jnp.float32),
            pltpu.VMEM((_CCH, H), jnp.float32),
            pltpu.VMEM((_CCH,), jnp.int32),
            pltpu.VMEM((_CCH,), jnp.int32),
            pltpu.VMEM((_CCH,), jnp.int32),
            pltpu.VMEM((_CCH,), jnp.int32),
            pltpu.VMEM((_CCH,), jnp.float32),
            pltpu.VMEM((_CCH,), jnp.float32),
            pltpu.SemaphoreType.DMA,
        ],
    )
    def k(os_hbm, posA_hbm, posB_hbm, wA_hbm, wB_hbm, fin_hbm,
          bufa0, bufa1, bufb0, bufb1, ia0, ia1, ib0, ib1, wa_v, wb_v, sg):
        wid = lax.axis_index("s") * 2 + lax.axis_index("c")
        t0 = wid * per_t
        bufas = (bufa0, bufa1)
        bufbs = (bufb0, bufb1)
        ias = (ia0, ia1)
        ibs = (ib0, ib1)
        ga = [None] * nch
        gb = [None] * nch
        pltpu.sync_copy(posA_hbm.at[pl.ds(t0, _CCH)], ia0)
        pltpu.sync_copy(posB_hbm.at[pl.ds(t0, _CCH)], ib0)
        ga[0] = pltpu.async_copy(os_hbm.at[ia0], bufa0, sg)
        gb[0] = pltpu.async_copy(os_hbm.at[ib0], bufb0, sg)
        for c in range(nch):
            if c + 1 < nch:
                nb = t0 + (c + 1) * _CCH
                pltpu.sync_copy(posA_hbm.at[pl.ds(nb, _CCH)],
                                ias[(c + 1) % 2])
                pltpu.sync_copy(posB_hbm.at[pl.ds(nb, _CCH)],
                                ibs[(c + 1) % 2])
                ga[c + 1] = pltpu.async_copy(
                    os_hbm.at[ias[(c + 1) % 2]], bufas[(c + 1) % 2], sg)
                gb[c + 1] = pltpu.async_copy(
                    os_hbm.at[ibs[(c + 1) % 2]], bufbs[(c + 1) % 2], sg)
            pltpu.sync_copy(wA_hbm.at[pl.ds(t0 + c * _CCH, _CCH)], wa_v)
            pltpu.sync_copy(wB_hbm.at[pl.ds(t0 + c * _CCH, _CCH)], wb_v)
            ga[c].wait()
            gb[c].wait()
            a = bufas[c % 2]
            b = bufbs[c % 2]

            def body(r, _):
                wa = wa_v[pl.ds(r, 1)]
                wb = wb_v[pl.ds(r, 1)]
                a[r, :] = a[r, :] * wa + b[r, :] * wb
                return 0

            lax.fori_loop(0, _CCH, body, 0)
            pltpu.sync_copy(a, fin_hbm.at[pl.ds(t0 + c * _CCH, _CCH)])

    return k(out_sorted, posA, posB, wA, wB)


def kernel(x, Wg, bg, We, be):
    B, S, H = x.shape
    h = x.reshape(-1, H)
    T = h.shape[0]
    E = Wg.shape[0]
    A = T * TOPK
    bg2 = bg.reshape(1, E)

    logits, idx, wn, within, counts = _route(h, Wg, bg2, T, H, E)

    counts_i = counts.reshape(E).astype(jnp.int32)
    off = jnp.concatenate([jnp.zeros((1,), jnp.int32),
                           jnp.cumsum(counts_i)])

    idx_f = idx.reshape(A)
    within_f = within.reshape(A)
    onehot_e = (idx_f[:, None] == jnp.arange(E, dtype=jnp.int32)[None, :])
    pos = within_f + jnp.sum(
        onehot_e.astype(jnp.int32) * off[:-1][None, :], axis=1)
    pos2 = pos.reshape(T, TOPK)
    posA = pos2[:, 0]
    posB = pos2[:, 1]
    hs = _dispatch_sc(h, posA, posB, T, H, A)

    tbl = _step_table(off, A)
    out_sorted = _ragged_mm(tbl, hs, We.astype(jnp.bfloat16), be, A, H, E)

    final = _combine_sc(out_sorted, posA, posB, wn[:, 0], wn[:, 1], T, H)

    return final.reshape(B, S, H), logits


# speedup vs baseline: 1.3581x; 1.3581x over previous
import functools

import jax
import jax.numpy as jnp
from jax import lax
from jax.experimental import pallas as pl
from jax.experimental.pallas import tpu as pltpu
from jax.experimental.pallas import tpu_sc as plsc

EXPERTS = 8
TOPK = 2
TB1 = 256
SB = 256
STEPS = 40


def _route_body(x_ref, wg_ref, bg_ref, ltri_ref, logits_ref, idx_ref,
                wn_ref, within_ref, counts_ref, carry_ref):
    b = pl.program_id(0)
    E = EXPERTS

    @pl.when(b == 0)
    def _():
        carry_ref[...] = jnp.zeros_like(carry_ref)

    x = x_ref[...]
    logits = jax.lax.dot_general(
        x, wg_ref[...],
        dimension_numbers=(((1,), (1,)), ((), ())),
        preferred_element_type=jnp.float32,
    ) + bg_ref[...]
    logits_ref[...] = logits

    la = jnp.reshape(jax.lax.broadcast_in_dim(
        logits, (TB1, E, E), (0, 1)), (TB1, E * E))
    lb = jnp.reshape(jax.lax.broadcast_in_dim(
        logits, (TB1, E, E), (0, 2)), (TB1, E * E))
    c_i = jax.lax.broadcasted_iota(jnp.int32, (TB1, E * E), 1)
    j_lt_e = (c_i // E) < (c_i % E)
    beats = jnp.where(
        (la > lb) | ((la == lb) & j_lt_e), 1.0, 0.0)
    s_r = jax.lax.broadcasted_iota(jnp.int32, (E * E, E), 0)
    s_c = jax.lax.broadcasted_iota(jnp.int32, (E * E, E), 1)
    sel = ((s_r % E) == s_c).astype(jnp.float32)
    rank = jax.lax.dot_general(
        beats, sel,
        dimension_numbers=(((1,), (0,)), ((), ())),
        preferred_element_type=jnp.float32,
    )

    oh0 = (rank == 0.0).astype(jnp.float32)
    oh1 = (rank == 1.0).astype(jnp.float32)
    l0 = jnp.sum(logits * oh0, axis=1, keepdims=True)
    l1 = jnp.sum(logits * oh1, axis=1, keepdims=True)
    w0 = 1.0 / (1.0 + jnp.exp(l1 - l0))
    lane = jax.lax.broadcasted_iota(
        jnp.int32, (TB1, E), 1).astype(jnp.float32)
    i0 = jnp.sum(lane * oh0, axis=1, keepdims=True)
    i1 = jnp.sum(lane * oh1, axis=1, keepdims=True)

    idx_ref[...] = jnp.concatenate([i0, i1], axis=1).astype(jnp.int32)
    wn_ref[...] = jnp.concatenate([w0, 1.0 - w0], axis=1)

    onehot = jnp.concatenate([oh0, oh1], axis=0)
    prefix = jax.lax.dot_general(
        ltri_ref[...], onehot,
        dimension_numbers=(((1,), (0,)), ((), ())),
        preferred_element_type=jnp.float32,
    ) + carry_ref[...]
    within = jnp.sum(onehot * prefix, axis=1, keepdims=True)
    within_ref[...] = jnp.concatenate(
        [within[:TB1], within[TB1:]], axis=1).astype(jnp.int32)

    carry_ref[...] += jnp.sum(onehot, axis=0, keepdims=True)
    counts_ref[...] = carry_ref[...]


def _ragged_mm_body(tbl_ref, hs_ref, we_ref, be_ref, out_ref):
    s = pl.program_id(0)
    e = tbl_ref[1, s]
    r0 = tbl_ref[2, s]
    r1 = tbl_ref[3, s]
    init = tbl_ref[4, s]
    tile = tbl_ref[0, s]

    w_e = jnp.reshape(we_ref[pl.ds(e, 1), :, :], (we_ref.shape[1],
                                                  we_ref.shape[2]))
    b_e = be_ref[pl.ds(e, 1), :]
    y = jax.lax.dot_general(
        hs_ref[...], w_e,
        dimension_numbers=(((1,), (1,)), ((), ())),
        preferred_element_type=jnp.float32,
    ) + b_e
    row = tile * SB + jax.lax.broadcasted_iota(jnp.int32, (SB, 1), 0)
    mask = jnp.logical_and(row >= r0, row < r1)
    contrib = jnp.where(mask, y, 0.0)

    @pl.when(init == 1)
    def _():
        out_ref[...] = contrib

    @pl.when(init != 1)
    def _():
        out_ref[...] += contrib


def _route(h, Wg, bg2, ltri, T, H, E):
    nb = T // TB1
    n2 = TOPK * TB1
    return pl.pallas_call(
        _route_body,
        grid=(nb,),
        in_specs=[
            pl.BlockSpec((TB1, H), lambda b: (b, 0)),
            pl.BlockSpec((E, H), lambda b: (0, 0)),
            pl.BlockSpec((1, E), lambda b: (0, 0)),
            pl.BlockSpec((n2, n2), lambda b: (0, 0)),
        ],
        out_specs=[
            pl.BlockSpec((TB1, E), lambda b: (b, 0)),
            pl.BlockSpec((TB1, TOPK), lambda b: (b, 0)),
            pl.BlockSpec((TB1, TOPK), lambda b: (b, 0)),
            pl.BlockSpec((TB1, TOPK), lambda b: (b, 0)),
            pl.BlockSpec((1, E), lambda b: (0, 0)),
        ],
        out_shape=[
            jax.ShapeDtypeStruct((T, E), jnp.float32),
            jax.ShapeDtypeStruct((T, TOPK), jnp.int32),
            jax.ShapeDtypeStruct((T, TOPK), jnp.float32),
            jax.ShapeDtypeStruct((T, TOPK), jnp.int32),
            jax.ShapeDtypeStruct((1, E), jnp.float32),
        ],
        scratch_shapes=[pltpu.VMEM((1, E), jnp.float32)],
        compiler_params=pltpu.CompilerParams(
            dimension_semantics=("arbitrary",)),
    )(h, Wg, bg2, ltri)


def _step_table(off9, A):
    nt = A // SB
    t_ids = jnp.arange(nt, dtype=jnp.int32)[:, None]
    lo = off9[:-1][None, :]
    hi = off9[1:][None, :]
    r0 = jnp.maximum(lo, t_ids * SB)
    r1 = jnp.minimum(hi, (t_ids + 1) * SB)
    active = (r1 > r0).reshape(-1)
    e_f = jnp.broadcast_to(jnp.arange(EXPERTS, dtype=jnp.int32)[None, :],
                           (nt, EXPERTS)).reshape(-1)
    t_f = jnp.broadcast_to(t_ids, (nt, EXPERTS)).reshape(-1)
    r0_f = r0.reshape(-1)
    r1_f = r1.reshape(-1)
    init_f = jnp.logical_and(active, r0_f == t_f * SB).astype(jnp.int32)

    dest = jnp.cumsum(active.astype(jnp.int32)) - active.astype(jnp.int32)
    dest = jnp.where(active, dest, STEPS - 1)
    pad = jnp.array([nt - 1, 0, 0, 0, 0], dtype=jnp.int32)
    vals = jnp.stack([t_f, e_f, r0_f, r1_f, init_f], axis=0)
    vals = jnp.where(active[None, :], vals, pad[:, None])
    tbl = jnp.zeros((5, STEPS), dtype=jnp.int32)
    tbl = tbl.at[0, :].set(nt - 1)
    tbl = tbl.at[:, dest].set(vals)
    return tbl


def _ragged_mm(tbl, hs, We, be, A, H, E):
    grid_spec = pltpu.PrefetchScalarGridSpec(
        num_scalar_prefetch=1,
        grid=(STEPS,),
        in_specs=[
            pl.BlockSpec((SB, H), lambda s, tbl: (tbl[0, s], 0)),
            pl.BlockSpec((E, H, H), lambda s, tbl: (0, 0, 0)),
            pl.BlockSpec((E, H), lambda s, tbl: (0, 0)),
        ],
        out_specs=pl.BlockSpec((SB, H), lambda s, tbl: (tbl[0, s], 0)),
    )
    return pl.pallas_call(
        _ragged_mm_body,
        grid_spec=grid_spec,
        out_shape=jax.ShapeDtypeStruct((A, H), jnp.float32),
        compiler_params=pltpu.CompilerParams(
            dimension_semantics=("arbitrary",)),
    )(tbl, hs, We, be)


_NW = 32
_DCH = 64
_CCH = 32


def _dispatch_sc(h, posA, posB, T, H, A):
    per_t = T // _NW
    nch = per_t // _DCH
    mesh = plsc.VectorSubcoreMesh(core_axis_name="c", subcore_axis_name="s")

    @functools.partial(
        pl.kernel, mesh=mesh,
        out_type=jax.ShapeDtypeStruct((A, H), jnp.float32),
        scratch_types=[
            pltpu.VMEM((_DCH, H), jnp.float32),
            pltpu.VMEM((_DCH, H), jnp.float32),
            pltpu.VMEM((_DCH,), jnp.int32),
            pltpu.VMEM((_DCH,), jnp.int32),
            pltpu.VMEM((_DCH,), jnp.int32),
            pltpu.VMEM((_DCH,), jnp.int32),
            pltpu.SemaphoreType.DMA,
            pltpu.SemaphoreType.DMA,
        ],
    )
    def k(h_hbm, posA_hbm, posB_hbm, hs_hbm,
          buf0, buf1, ia0, ia1, ib0, ib1, sg, ss):
        wid = lax.axis_index("s") * 2 + lax.axis_index("c")
        t0 = wid * per_t
        bufs = (buf0, buf1)
        ias = (ia0, ia1)
        ibs = (ib0, ib1)
        gets = [None] * nch
        gets[0] = pltpu.async_copy(h_hbm.at[pl.ds(t0, _DCH)], bufs[0], sg)
        puts = []
        for c in range(nch):
            base = t0 + c * _DCH
            pltpu.sync_copy(posA_hbm.at[pl.ds(base, _DCH)], ias[c % 2])
            pltpu.sync_copy(posB_hbm.at[pl.ds(base, _DCH)], ibs[c % 2])
            gets[c].wait()
            if c + 1 < nch:
                gets[c + 1] = pltpu.async_copy(
                    h_hbm.at[pl.ds(t0 + (c + 1) * _DCH, _DCH)],
                    bufs[(c + 1) % 2], sg)
            puts.append(pltpu.async_copy(
                bufs[c % 2], hs_hbm.at[ias[c % 2]], ss))
            puts.append(pltpu.async_copy(
                bufs[c % 2], hs_hbm.at[ibs[c % 2]], ss))
        for hdl in puts:
            hdl.wait()

    return k(h, posA, posB)


def _combine_sc(out_sorted, posA, posB, wA, wB, T, H):
    per_t = T // _NW
    nch = per_t // _CCH
    mesh = plsc.VectorSubcoreMesh(core_axis_name="c", subcore_axis_name="s")

    @functools.partial(
        pl.kernel, mesh=mesh,
        out_type=jax.ShapeDtypeStruct((T, H), jnp.float32),
        scratch_types=[
            pltpu.VMEM((_CCH, H), jnp.float32),
            pltpu.VMEM((_CCH, H), jnp.float32),
            pltpu.VMEM((_CCH, H), jnp.float32),
            pltpu.VMEM((_CCH, H), jnp.float32),
            pltpu.VMEM((_CCH,), jnp.int32),
            pltpu.VMEM((_CCH,), jnp.int32),
            pltpu.VMEM((_CCH,), jnp.int32),
            pltpu.VMEM((_CCH,), jnp.int32),
            pltpu.VMEM((_CCH,), jnp.float32),
            pltpu.VMEM((_CCH,), jnp.float32),
            pltpu.SemaphoreType.DMA,
        ],
    )
    def k(os_hbm, posA_hbm, posB_hbm, wA_hbm, wB_hbm, fin_hbm,
          bufa0, bufa1, bufb0, bufb1, ia0, ia1, ib0, ib1, wa_v, wb_v, sg):
        wid = lax.axis_index("s") * 2 + lax.axis_index("c")
        t0 = wid * per_t
        bufas = (bufa0, bufa1)
        bufbs = (bufb0, bufb1)
        ias = (ia0, ia1)
        ibs = (ib0, ib1)
        ga = [None] * nch
        gb = [None] * nch
        pltpu.sync_copy(posA_hbm.at[pl.ds(t0, _CCH)], ia0)
        pltpu.sync_copy(posB_hbm.at[pl.ds(t0, _CCH)], ib0)
        ga[0] = pltpu.async_copy(os_hbm.at[ia0], bufa0, sg)
        gb[0] = pltpu.async_copy(os_hbm.at[ib0], bufb0, sg)
        for c in range(nch):
            if c + 1 < nch:
                nb = t0 + (c + 1) * _CCH
                pltpu.sync_copy(posA_hbm.at[pl.ds(nb, _CCH)],
                                ias[(c + 1) % 2])
                pltpu.sync_copy(posB_hbm.at[pl.ds(nb, _CCH)],
                                ibs[(c + 1) % 2])
                ga[c + 1] = pltpu.async_copy(
                    os_hbm.at[ias[(c + 1) % 2]], bufas[(c + 1) % 2], sg)
                gb[c + 1] = pltpu.async_copy(
                    os_hbm.at[ibs[(c + 1) % 2]], bufbs[(c + 1) % 2], sg)
            pltpu.sync_copy(wA_hbm.at[pl.ds(t0 + c * _CCH, _CCH)], wa_v)
            pltpu.sync_copy(wB_hbm.at[pl.ds(t0 + c * _CCH, _CCH)], wb_v)
            ga[c].wait()
            gb[c].wait()
            a = bufas[c % 2]
            b = bufbs[c % 2]

            def body(r, _):
                wa = wa_v[pl.ds(r, 1)]
                wb = wb_v[pl.ds(r, 1)]
                a[r, :] = a[r, :] * wa + b[r, :] * wb
                return 0

            lax.fori_loop(0, _CCH, body, 0)
            pltpu.sync_copy(a, fin_hbm.at[pl.ds(t0 + c * _CCH, _CCH)])

    return k(out_sorted, posA, posB, wA, wB)


def kernel(x, Wg, bg, We, be):
    B, S, H = x.shape
    h = x.reshape(-1, H)
    T = h.shape[0]
    E = Wg.shape[0]
    A = T * TOPK
    bg2 = bg.reshape(1, E)

    n2 = TOPK * TB1
    ltri = (jnp.arange(n2, dtype=jnp.int32)[:, None]
            > jnp.arange(n2, dtype=jnp.int32)[None, :]).astype(jnp.float32)
    logits, idx, wn, within, counts = _route(h, Wg, bg2, ltri, T, H, E)

    counts_i = counts.reshape(E).astype(jnp.int32)
    off = jnp.concatenate([jnp.zeros((1,), jnp.int32),
                           jnp.cumsum(counts_i)])

    idx_f = idx.reshape(A)
    within_f = within.reshape(A)
    onehot_e = (idx_f[:, None] == jnp.arange(E, dtype=jnp.int32)[None, :])
    pos = within_f + jnp.sum(
        onehot_e.astype(jnp.int32) * off[:-1][None, :], axis=1)
    pos2 = pos.reshape(T, TOPK)
    posA = pos2[:, 0]
    posB = pos2[:, 1]
    hs = _dispatch_sc(h, posA, posB, T, H, A)

    tbl = _step_table(off, A)
    out_sorted = _ragged_mm(tbl, hs, We, be, A, H, E)

    final = _combine_sc(out_sorted, posA, posB, wn[:, 0], wn[:, 1], T, H)

    return final.reshape(B, S, H), logits


# speedup vs baseline: 1.3607x; 1.0019x over previous
import functools

import jax
import jax.numpy as jnp
from jax import lax
from jax.experimental import pallas as pl
from jax.experimental.pallas import tpu as pltpu
from jax.experimental.pallas import tpu_sc as plsc

EXPERTS = 8
TOPK = 2
TB1 = 256
SB = 256
STEPS = 40


def _route_body(x_ref, wg_ref, bg_ref, ltri_ref, jlt_ref, sel_ref,
                logits_ref, idx_ref, wn_ref, within_ref, counts_ref,
                carry_ref):
    b = pl.program_id(0)
    E = EXPERTS

    @pl.when(b == 0)
    def _():
        carry_ref[...] = jnp.zeros_like(carry_ref)

    x = x_ref[...]
    logits = jax.lax.dot_general(
        x, wg_ref[...],
        dimension_numbers=(((1,), (1,)), ((), ())),
        preferred_element_type=jnp.float32,
    ) + bg_ref[...]
    logits_ref[...] = logits

    la = jnp.reshape(jax.lax.broadcast_in_dim(
        logits, (TB1, E, E), (0, 1)), (TB1, E * E))
    lb = jnp.reshape(jax.lax.broadcast_in_dim(
        logits, (TB1, E, E), (0, 2)), (TB1, E * E))
    beats = jnp.where(la > lb, 1.0,
                      jnp.where(la == lb, jlt_ref[...], 0.0))
    rank = jax.lax.dot_general(
        beats, sel_ref[...],
        dimension_numbers=(((1,), (0,)), ((), ())),
        preferred_element_type=jnp.float32,
    )

    oh0 = (rank == 0.0).astype(jnp.float32)
    oh1 = (rank == 1.0).astype(jnp.float32)
    l0 = jnp.sum(logits * oh0, axis=1, keepdims=True)
    l1 = jnp.sum(logits * oh1, axis=1, keepdims=True)
    w0 = 1.0 / (1.0 + jnp.exp(l1 - l0))
    lane = jax.lax.broadcasted_iota(
        jnp.int32, (TB1, E), 1).astype(jnp.float32)
    i0 = jnp.sum(lane * oh0, axis=1, keepdims=True)
    i1 = jnp.sum(lane * oh1, axis=1, keepdims=True)

    idx_ref[...] = jnp.concatenate([i0, i1], axis=1).astype(jnp.int32)
    wn_ref[...] = jnp.concatenate([w0, 1.0 - w0], axis=1)

    s0 = jnp.sum(oh0, axis=0, keepdims=True)
    p0 = jax.lax.dot_general(
        ltri_ref[...], oh0,
        dimension_numbers=(((1,), (0,)), ((), ())),
        preferred_element_type=jnp.float32,
    ) + carry_ref[...]
    p1 = jax.lax.dot_general(
        ltri_ref[...], oh1,
        dimension_numbers=(((1,), (0,)), ((), ())),
        preferred_element_type=jnp.float32,
    ) + (carry_ref[...] + s0)
    w_in0 = jnp.sum(oh0 * p0, axis=1, keepdims=True)
    w_in1 = jnp.sum(oh1 * p1, axis=1, keepdims=True)
    within_ref[...] = jnp.concatenate(
        [w_in0, w_in1], axis=1).astype(jnp.int32)

    carry_ref[...] += s0 + jnp.sum(oh1, axis=0, keepdims=True)
    counts_ref[...] = carry_ref[...]


def _ragged_mm_body(tbl_ref, hs_ref, we_ref, be_ref, out_ref):
    s = pl.program_id(0)
    e = tbl_ref[1, s]
    r0 = tbl_ref[2, s]
    r1 = tbl_ref[3, s]
    init = tbl_ref[4, s]
    tile = tbl_ref[0, s]

    w_e = jnp.reshape(we_ref[pl.ds(e, 1), :, :], (we_ref.shape[1],
                                                  we_ref.shape[2]))
    b_e = be_ref[pl.ds(e, 1), :]
    y = jax.lax.dot_general(
        hs_ref[...], w_e,
        dimension_numbers=(((1,), (1,)), ((), ())),
        preferred_element_type=jnp.float32,
    ) + b_e
    row = tile * SB + jax.lax.broadcasted_iota(jnp.int32, (SB, 1), 0)
    mask = jnp.logical_and(row >= r0, row < r1)
    contrib = jnp.where(mask, y, 0.0)

    @pl.when(init == 1)
    def _():
        out_ref[...] = contrib

    @pl.when(init != 1)
    def _():
        out_ref[...] += contrib


def _route(h, Wg, bg2, ltri, jlt, sel, T, H, E):
    nb = T // TB1
    return pl.pallas_call(
        _route_body,
        grid=(nb,),
        in_specs=[
            pl.BlockSpec((TB1, H), lambda b: (b, 0)),
            pl.BlockSpec((E, H), lambda b: (0, 0)),
            pl.BlockSpec((1, E), lambda b: (0, 0)),
            pl.BlockSpec((TB1, TB1), lambda b: (0, 0)),
            pl.BlockSpec((1, E * E), lambda b: (0, 0)),
            pl.BlockSpec((E * E, E), lambda b: (0, 0)),
        ],
        out_specs=[
            pl.BlockSpec((TB1, E), lambda b: (b, 0)),
            pl.BlockSpec((TB1, TOPK), lambda b: (b, 0)),
            pl.BlockSpec((TB1, TOPK), lambda b: (b, 0)),
            pl.BlockSpec((TB1, TOPK), lambda b: (b, 0)),
            pl.BlockSpec((1, E), lambda b: (0, 0)),
        ],
        out_shape=[
            jax.ShapeDtypeStruct((T, E), jnp.float32),
            jax.ShapeDtypeStruct((T, TOPK), jnp.int32),
            jax.ShapeDtypeStruct((T, TOPK), jnp.float32),
            jax.ShapeDtypeStruct((T, TOPK), jnp.int32),
            jax.ShapeDtypeStruct((1, E), jnp.float32),
        ],
        scratch_shapes=[pltpu.VMEM((1, E), jnp.float32)],
        compiler_params=pltpu.CompilerParams(
            dimension_semantics=("arbitrary",)),
    )(h, Wg, bg2, ltri, jlt, sel)


def _step_table(off9, A):
    nt = A // SB
    t_ids = jnp.arange(nt, dtype=jnp.int32)[:, None]
    lo = off9[:-1][None, :]
    hi = off9[1:][None, :]
    r0 = jnp.maximum(lo, t_ids * SB)
    r1 = jnp.minimum(hi, (t_ids + 1) * SB)
    active = (r1 > r0).reshape(-1)
    e_f = jnp.broadcast_to(jnp.arange(EXPERTS, dtype=jnp.int32)[None, :],
                           (nt, EXPERTS)).reshape(-1)
    t_f = jnp.broadcast_to(t_ids, (nt, EXPERTS)).reshape(-1)
    r0_f = r0.reshape(-1)
    r1_f = r1.reshape(-1)
    init_f = jnp.logical_and(active, r0_f == t_f * SB).astype(jnp.int32)

    dest = jnp.cumsum(active.astype(jnp.int32)) - active.astype(jnp.int32)
    dest = jnp.where(active, dest, STEPS - 1)
    pad = jnp.array([nt - 1, 0, 0, 0, 0], dtype=jnp.int32)
    vals = jnp.stack([t_f, e_f, r0_f, r1_f, init_f], axis=0)
    vals = jnp.where(active[None, :], vals, pad[:, None])
    tbl = jnp.zeros((5, STEPS), dtype=jnp.int32)
    tbl = tbl.at[0, :].set(nt - 1)
    tbl = tbl.at[:, dest].set(vals)
    return tbl


def _ragged_mm(tbl, hs, We, be, A, H, E):
    grid_spec = pltpu.PrefetchScalarGridSpec(
        num_scalar_prefetch=1,
        grid=(STEPS,),
        in_specs=[
            pl.BlockSpec((SB, H), lambda s, tbl: (tbl[0, s], 0)),
            pl.BlockSpec((E, H, H), lambda s, tbl: (0, 0, 0)),
            pl.BlockSpec((E, H), lambda s, tbl: (0, 0)),
        ],
        out_specs=pl.BlockSpec((SB, H), lambda s, tbl: (tbl[0, s], 0)),
    )
    return pl.pallas_call(
        _ragged_mm_body,
        grid_spec=grid_spec,
        out_shape=jax.ShapeDtypeStruct((A, H), jnp.float32),
        compiler_params=pltpu.CompilerParams(
            dimension_semantics=("arbitrary",)),
    )(tbl, hs, We, be)


_NW = 32
_DCH = 64
_CCH = 32


def _dispatch_sc(h, posA, posB, T, H, A):
    per_t = T // _NW
    nch = per_t // _DCH
    mesh = plsc.VectorSubcoreMesh(core_axis_name="c", subcore_axis_name="s")

    @functools.partial(
        pl.kernel, mesh=mesh,
        out_type=jax.ShapeDtypeStruct((A, H), jnp.float32),
        scratch_types=[
            pltpu.VMEM((_DCH, H), jnp.float32),
            pltpu.VMEM((_DCH, H), jnp.float32),
            pltpu.VMEM((_DCH,), jnp.int32),
            pltpu.VMEM((_DCH,), jnp.int32),
            pltpu.VMEM((_DCH,), jnp.int32),
            pltpu.VMEM((_DCH,), jnp.int32),
            pltpu.SemaphoreType.DMA,
            pltpu.SemaphoreType.DMA,
        ],
    )
    def k(h_hbm, posA_hbm, posB_hbm, hs_hbm,
          buf0, buf1, ia0, ia1, ib0, ib1, sg, ss):
        wid = lax.axis_index("s") * 2 + lax.axis_index("c")
        t0 = wid * per_t
        bufs = (buf0, buf1)
        ias = (ia0, ia1)
        ibs = (ib0, ib1)
        gets = [None] * nch
        gets[0] = pltpu.async_copy(h_hbm.at[pl.ds(t0, _DCH)], bufs[0], sg)
        puts = []
        for c in range(nch):
            base = t0 + c * _DCH
            pltpu.sync_copy(posA_hbm.at[pl.ds(base, _DCH)], ias[c % 2])
            pltpu.sync_copy(posB_hbm.at[pl.ds(base, _DCH)], ibs[c % 2])
            gets[c].wait()
            if c + 1 < nch:
                gets[c + 1] = pltpu.async_copy(
                    h_hbm.at[pl.ds(t0 + (c + 1) * _DCH, _DCH)],
                    bufs[(c + 1) % 2], sg)
            puts.append(pltpu.async_copy(
                bufs[c % 2], hs_hbm.at[ias[c % 2]], ss))
            puts.append(pltpu.async_copy(
                bufs[c % 2], hs_hbm.at[ibs[c % 2]], ss))
        for hdl in puts:
            hdl.wait()

    return k(h, posA, posB)


def _combine_sc(out_sorted, posA, posB, wA, wB, T, H):
    per_t = T // _NW
    nch = per_t // _CCH
    mesh = plsc.VectorSubcoreMesh(core_axis_name="c", subcore_axis_name="s")

    @functools.partial(
        pl.kernel, mesh=mesh,
        out_type=jax.ShapeDtypeStruct((T, H), jnp.float32),
        scratch_types=[
            pltpu.VMEM((_CCH, H), jnp.float32),
            pltpu.VMEM((_CCH, H), jnp.float32),
            pltpu.VMEM((_CCH, H), jnp.float32),
            pltpu.VMEM((_CCH, H), jnp.float32),
            pltpu.VMEM((_CCH,), jnp.int32),
            pltpu.VMEM((_CCH,), jnp.int32),
            pltpu.VMEM((_CCH,), jnp.int32),
            pltpu.VMEM((_CCH,), jnp.int32),
            pltpu.VMEM((_CCH,), jnp.float32),
            pltpu.VMEM((_CCH,), jnp.float32),
            pltpu.SemaphoreType.DMA,
        ],
    )
    def k(os_hbm, posA_hbm, posB_hbm, wA_hbm, wB_hbm, fin_hbm,
          bufa0, bufa1, bufb0, bufb1, ia0, ia1, ib0, ib1, wa_v, wb_v, sg):
        wid = lax.axis_index("s") * 2 + lax.axis_index("c")
        t0 = wid * per_t
        bufas = (bufa0, bufa1)
        bufbs = (bufb0, bufb1)
        ias = (ia0, ia1)
        ibs = (ib0, ib1)
        ga = [None] * nch
        gb = [None] * nch
        pltpu.sync_copy(posA_hbm.at[pl.ds(t0, _CCH)], ia0)
        pltpu.sync_copy(posB_hbm.at[pl.ds(t0, _CCH)], ib0)
        ga[0] = pltpu.async_copy(os_hbm.at[ia0], bufa0, sg)
        gb[0] = pltpu.async_copy(os_hbm.at[ib0], bufb0, sg)
        for c in range(nch):
            if c + 1 < nch:
                nb = t0 + (c + 1) * _CCH
                pltpu.sync_copy(posA_hbm.at[pl.ds(nb, _CCH)],
                                ias[(c + 1) % 2])
                pltpu.sync_copy(posB_hbm.at[pl.ds(nb, _CCH)],
                                ibs[(c + 1) % 2])
                ga[c + 1] = pltpu.async_copy(
                    os_hbm.at[ias[(c + 1) % 2]], bufas[(c + 1) % 2], sg)
                gb[c + 1] = pltpu.async_copy(
                    os_hbm.at[ibs[(c + 1) % 2]], bufbs[(c + 1) % 2], sg)
            pltpu.sync_copy(wA_hbm.at[pl.ds(t0 + c * _CCH, _CCH)], wa_v)
            pltpu.sync_copy(wB_hbm.at[pl.ds(t0 + c * _CCH, _CCH)], wb_v)
            ga[c].wait()
            gb[c].wait()
            a = bufas[c % 2]
            b = bufbs[c % 2]

            def body(r, _):
                wa = wa_v[pl.ds(r, 1)]
                wb = wb_v[pl.ds(r, 1)]
                a[r, :] = a[r, :] * wa + b[r, :] * wb
                return 0

            lax.fori_loop(0, _CCH, body, 0)
            pltpu.sync_copy(a, fin_hbm.at[pl.ds(t0 + c * _CCH, _CCH)])

    return k(out_sorted, posA, posB, wA, wB)


def kernel(x, Wg, bg, We, be):
    B, S, H = x.shape
    h = x.reshape(-1, H)
    T = h.shape[0]
    E = Wg.shape[0]
    A = T * TOPK
    bg2 = bg.reshape(1, E)

    ltri = (jnp.arange(TB1, dtype=jnp.int32)[:, None]
            > jnp.arange(TB1, dtype=jnp.int32)[None, :]).astype(jnp.float32)
    kk = jnp.arange(E * E, dtype=jnp.int32)
    jlt = (kk // E < kk % E).astype(jnp.float32).reshape(1, E * E)
    sel = ((kk % E)[:, None]
           == jnp.arange(E, dtype=jnp.int32)[None, :]).astype(jnp.float32)
    logits, idx, wn, within, counts = _route(
        h, Wg, bg2, ltri, jlt, sel, T, H, E)

    counts_i = counts.reshape(E).astype(jnp.int32)
    off = jnp.concatenate([jnp.zeros((1,), jnp.int32),
                           jnp.cumsum(counts_i)])

    idx_f = idx.reshape(A)
    within_f = within.reshape(A)
    onehot_e = (idx_f[:, None] == jnp.arange(E, dtype=jnp.int32)[None, :])
    pos = within_f + jnp.sum(
        onehot_e.astype(jnp.int32) * off[:-1][None, :], axis=1)
    pos2 = pos.reshape(T, TOPK)
    posA = pos2[:, 0]
    posB = pos2[:, 1]
    hs = _dispatch_sc(h, posA, posB, T, H, A)

    tbl = _step_table(off, A)
    out_sorted = _ragged_mm(tbl, hs, We, be, A, H, E)

    final = _combine_sc(out_sorted, posA, posB, wn[:, 0], wn[:, 1], T, H)

    return final.reshape(B, S, H), logits
